# baseline (speedup 1.0000x reference)
import jax
import jax.numpy as jnp
from jax.experimental import pallas as pl
from jax.experimental.pallas import tpu as pltpu

MAX_NUM_TILES = 4
HIDDEN_SIZE = 1280
NUM_PATCHES = 1025


def _body(ids_ref, gate_ref, table_ref, hid_ref, out_ref):
    b = pl.program_id(0)
    t = pl.program_id(1)
    row = ids_ref[b]
    emb = table_ref[row, pl.ds(t * HIDDEN_SIZE, HIDDEN_SIZE)]
    g = jnp.tanh(gate_ref[0])
    out_ref[...] = hid_ref[...] + (g * emb)[None, None, None, :]


def kernel(hidden_state, aspect_ratio_ids, embedding_table, gate):
    batch = hidden_state.shape[0]
    ids = aspect_ratio_ids.astype(jnp.int32)

    grid_spec = pltpu.PrefetchScalarGridSpec(
        num_scalar_prefetch=1,
        grid=(batch, MAX_NUM_TILES),
        in_specs=[
            pl.BlockSpec((1,), lambda b, t, ids: (0,),
                         memory_space=pltpu.SMEM),
            pl.BlockSpec(embedding_table.shape, lambda b, t, ids: (0, 0)),
            pl.BlockSpec((1, 1, NUM_PATCHES, HIDDEN_SIZE),
                         lambda b, t, ids: (b, t, 0, 0)),
        ],
        out_specs=pl.BlockSpec((1, 1, NUM_PATCHES, HIDDEN_SIZE),
                               lambda b, t, ids: (b, t, 0, 0)),
    )

    return pl.pallas_call(
        _body,
        grid_spec=grid_spec,
        out_shape=jax.ShapeDtypeStruct(hidden_state.shape, hidden_state.dtype),
    )(ids, gate, embedding_table, hidden_state)
